# baseline (speedup 1.0000x reference)
import functools
import math

import jax
import jax.numpy as jnp
from jax.experimental import pallas as pl
from jax.experimental.pallas import tpu as pltpu

_NEG_INF = float("-inf")


def _float_keys(x):
    bits = jax.lax.bitcast_convert_type(x, jnp.int32)
    sh = jax.lax.shift_right_arithmetic(bits, 31)
    return jax.lax.bitwise_xor(bits, jax.lax.bitwise_and(sh, jnp.int32(0x7FFFFFFF)))


def _topk_mask(x, k):
    key = _float_keys(x)
    cols = x.shape[1]
    cnt0 = jnp.sum((key >= 0).astype(jnp.int32), axis=0, keepdims=True)
    cur = jnp.where(cnt0 >= k, jnp.int32(0), jnp.int32(-2147483648))
    for bit in range(30, -1, -1):
        cand = jax.lax.bitwise_or(cur, jnp.int32(1 << bit))
        cnt = jnp.sum((key >= cand).astype(jnp.int32), axis=0, keepdims=True)
        cur = jnp.where(cnt >= k, cand, cur)
    return key >= cur


def _mask_softmax_ent(logits, mask):
    m = jnp.max(logits, axis=0, keepdims=True)
    t = logits - m
    e = jnp.where(mask, jnp.exp(t), 0.0)
    d = jnp.sum(e, axis=0, keepdims=True)
    c = e / d
    ent = jnp.log(d) - jnp.sum(jnp.where(mask, e * t, 0.0), axis=0, keepdims=True) / d
    return c, ent


def _nsum(x, ti, n_dim):
    return jnp.concatenate(
        [jnp.sum(x[:, n * ti:(n + 1) * ti], axis=1, keepdims=True)
         for n in range(n_dim)], axis=1)


def _csum(c, u, ti, n_dim):
    return jnp.concatenate(
        [jnp.sum(c * u[:, n * ti:(n + 1) * ti], axis=1, keepdims=True)
         for n in range(n_dim)], axis=1)


def _bup(u, v, ti, n_dim):
    return functools.reduce(
        lambda a, b: a + b,
        [u[:, n * ti:(n + 1) * ti] * v[:, n:n + 1] for n in range(n_dim)])


def _p1_body(n_dim, ti, u_ref, s_ref):
    t = pl.program_id(1)

    @pl.when(t == 0)
    def _():
        s_ref[...] = jnp.zeros(s_ref.shape, s_ref.dtype)

    u = u_ref[0, :, 0, 0, :]
    s_ref[0] += _nsum(u, ti, n_dim)


def _p2_body(k0, n_dim, ti, u_ref, v0_ref, s1_ref, ent_ref, bvec_ref):
    t = pl.program_id(1)

    @pl.when(t == 0)
    def _():
        s1_ref[...] = jnp.zeros(s1_ref.shape, s1_ref.dtype)
        ent_ref[...] = jnp.zeros(ent_ref.shape, ent_ref.dtype)

    u = u_ref[0, :, 0, 0, :]
    v0 = v0_ref[0]
    b_up = _bup(u, v0, ti, n_dim)
    mask = _topk_mask(b_up, k0)
    c, ent = _mask_softmax_ent(jnp.where(mask, b_up, _NEG_INF), mask)
    s1_ref[0] += _csum(c, u, ti, n_dim)
    ent_ref[0] += jnp.broadcast_to(ent, ent_ref.shape[1:])
    bvec_ref[0] = jnp.where(mask, b_up, _NEG_INF)


def _p3_body(k1, n_dim, ti, u_ref, v1_ref, bvec_ref, s2_ref, ent_ref):
    t = pl.program_id(1)

    @pl.when(t == 0)
    def _():
        s2_ref[...] = jnp.zeros(s2_ref.shape, s2_ref.dtype)
        ent_ref[...] = jnp.zeros(ent_ref.shape, ent_ref.dtype)

    u = u_ref[0, :, 0, 0, :]
    v1 = v1_ref[0]
    b2 = bvec_ref[0] + _bup(u, v1, ti, n_dim)
    mask = _topk_mask(b2, k1)
    c, ent = _mask_softmax_ent(jnp.where(mask, b2, _NEG_INF), mask)
    s2_ref[0] += _csum(c, u, ti, n_dim)
    ent_ref[0] += jnp.broadcast_to(ent, ent_ref.shape[1:])


def _squash_bias(s, bias):
    reset = jnp.sum(s, axis=2) == 0
    sb = jnp.where(reset[:, :, None], 0.0, s + bias)
    mag_sq = jnp.sum(sb * sb, axis=-1, keepdims=True)
    mag = jnp.sqrt(mag_sq + 1e-12)
    return (mag_sq / (1.0 + mag_sq)) * (sb / (mag + 1e-8))


def kernel(u_hat, iters, bias):
    del iters
    B, J, I, N = u_hat.shape
    TI = min(512, I)
    T = I // TI
    NTI = N * TI
    f32 = jnp.float32

    k0 = math.ceil(J * 0.5)
    k1 = math.ceil(k0 * 0.5)

    u4 = u_hat.reshape(B, J, T, TI, N).swapaxes(3, 4).reshape(B, J, T, 1, NTI)

    cparams = pltpu.CompilerParams(
        dimension_semantics=("parallel", "arbitrary"))

    s0 = pl.pallas_call(
        functools.partial(_p1_body, N, TI),
        grid=(B, T),
        in_specs=[pl.BlockSpec((1, J, 1, 1, NTI), lambda b, t: (b, 0, t, 0, 0))],
        out_specs=pl.BlockSpec((1, J, N), lambda b, t: (b, 0, 0)),
        out_shape=jax.ShapeDtypeStruct((B, J, N), f32),
        compiler_params=cparams,
    )(u4)

    v0 = _squash_bias(s0 * (1.0 / J), bias)

    s1, ent1, bvec1 = pl.pallas_call(
        functools.partial(_p2_body, k0, N, TI),
        grid=(B, T),
        in_specs=[
            pl.BlockSpec((1, J, 1, 1, NTI), lambda b, t: (b, 0, t, 0, 0)),
            pl.BlockSpec((1, J, N), lambda b, t: (b, 0, 0)),
        ],
        out_specs=[
            pl.BlockSpec((1, J, N), lambda b, t: (b, 0, 0)),
            pl.BlockSpec((1, 8, TI), lambda b, t: (b, 0, 0)),
            pl.BlockSpec((1, J, TI), lambda b, t: (b, 0, t)),
        ],
        out_shape=[
            jax.ShapeDtypeStruct((B, J, N), f32),
            jax.ShapeDtypeStruct((B, 8, TI), f32),
            jax.ShapeDtypeStruct((B, J, I), f32),
        ],
        compiler_params=cparams,
    )(u4, v0)

    v1 = _squash_bias(s1, bias)

    s2, ent2 = pl.pallas_call(
        functools.partial(_p3_body, k1, N, TI),
        grid=(B, T),
        in_specs=[
            pl.BlockSpec((1, J, 1, 1, NTI), lambda b, t: (b, 0, t, 0, 0)),
            pl.BlockSpec((1, J, N), lambda b, t: (b, 0, 0)),
            pl.BlockSpec((1, J, TI), lambda b, t: (b, 0, t)),
        ],
        out_specs=[
            pl.BlockSpec((1, J, N), lambda b, t: (b, 0, 0)),
            pl.BlockSpec((1, 8, TI), lambda b, t: (b, 0, 0)),
        ],
        out_shape=[
            jax.ShapeDtypeStruct((B, J, N), f32),
            jax.ShapeDtypeStruct((B, 8, TI), f32),
        ],
        compiler_params=cparams,
    )(u4, v1, bvec1)

    v2 = _squash_bias(s2, bias)

    e0 = jnp.full((B,), jnp.log(f32(J)), dtype=f32)
    e1 = jnp.sum(ent1[:, 0, :], axis=-1) * (1.0 / I)
    e2 = jnp.sum(ent2[:, 0, :], axis=-1) * (1.0 / I)
    entropy_layer = jnp.stack([e0, e1, e2], axis=1)
    return v2, entropy_layer


# speedup vs baseline: 1.4055x; 1.4055x over previous
import math

import jax
import jax.numpy as jnp
from jax.experimental import pallas as pl
from jax.experimental.pallas import tpu as pltpu

_NEG_INF = float("-inf")


def _float_keys(x):
    bits = jax.lax.bitcast_convert_type(x, jnp.int32)
    sh = jax.lax.shift_right_arithmetic(bits, 31)
    return jax.lax.bitwise_xor(bits, jax.lax.bitwise_and(sh, jnp.int32(0x7FFFFFFF)))


def _topk_mask(x, k):
    key = _float_keys(x)
    cnt0 = jnp.sum((key >= 0).astype(jnp.int32), axis=0, keepdims=True)
    cur = jnp.where(cnt0 >= k, jnp.int32(0), jnp.int32(-2147483648))
    for bit in range(30, 14, -1):
        cand = jax.lax.bitwise_or(cur, jnp.int32(1 << bit))
        cnt = jnp.sum((key >= cand).astype(jnp.int32), axis=0, keepdims=True)
        cur = jnp.where(cnt >= k, cand, cur)
    return key >= cur


def _mask_softmax_ent(x, mask):
    m = jnp.max(x, axis=0, keepdims=True)
    t = jnp.maximum(x - m, -100.0)
    e = jnp.where(mask, jnp.exp(t), 0.0)
    et = e * t
    d = jnp.sum(e, axis=0, keepdims=True)
    c = e / d
    ent = jnp.log(d) - jnp.sum(et, axis=0, keepdims=True) / d
    return c, ent


def _pass1_body(u_ref, r_ref, s_ref):
    t = pl.program_id(1)

    @pl.when(t == 0)
    def _():
        s_ref[...] = jnp.zeros(s_ref.shape, s_ref.dtype)

    u = u_ref[0]
    s_ref[0] += jnp.dot(u, r_ref[...], preferred_element_type=jnp.float32)


def _pass2_body(k0, u_ref, v0_ref, rt_ref, s_ref, st_ref, r_ref,
                s1_ref, ent_ref, bvec_ref):
    t = pl.program_id(1)

    @pl.when(t == 0)
    def _():
        s1_ref[...] = jnp.zeros(s1_ref.shape, s1_ref.dtype)
        ent_ref[...] = jnp.zeros(ent_ref.shape, ent_ref.dtype)

    u = u_ref[0]
    v0 = v0_ref[0]
    v0t = jnp.dot(v0, rt_ref[...], preferred_element_type=jnp.float32)
    b_up = jnp.dot(u * v0t, s_ref[...], preferred_element_type=jnp.float32)
    mask = _topk_mask(b_up, k0)
    bvec_ref[0] = jnp.where(mask, b_up, _NEG_INF)
    c, ent = _mask_softmax_ent(b_up, mask)
    ct = jnp.dot(c, st_ref[...], preferred_element_type=jnp.float32)
    s1_ref[0] += jnp.dot(ct * u, r_ref[...], preferred_element_type=jnp.float32)
    ent_ref[0] += jnp.broadcast_to(ent, ent_ref.shape[1:])


def _pass3_body(k1, u_ref, v1_ref, bvec_ref, rt_ref, s_ref, st_ref, r_ref,
                s2_ref, ent_ref):
    t = pl.program_id(1)

    @pl.when(t == 0)
    def _():
        s2_ref[...] = jnp.zeros(s2_ref.shape, s2_ref.dtype)
        ent_ref[...] = jnp.zeros(ent_ref.shape, ent_ref.dtype)

    u = u_ref[0]
    v1 = v1_ref[0]
    v1t = jnp.dot(v1, rt_ref[...], preferred_element_type=jnp.float32)
    b2 = bvec_ref[0] + jnp.dot(u * v1t, s_ref[...],
                               preferred_element_type=jnp.float32)
    mask = _topk_mask(b2, k1)
    c, ent = _mask_softmax_ent(b2, mask)
    ct = jnp.dot(c, st_ref[...], preferred_element_type=jnp.float32)
    s2_ref[0] += jnp.dot(ct * u, r_ref[...], preferred_element_type=jnp.float32)
    ent_ref[0] += jnp.broadcast_to(ent, ent_ref.shape[1:])


def _squash_bias(s, bias):
    reset = jnp.sum(s, axis=2) == 0
    sb = jnp.where(reset[:, :, None], 0.0, s + bias)
    mag_sq = jnp.sum(sb * sb, axis=-1, keepdims=True)
    mag = jnp.sqrt(mag_sq + 1e-12)
    return (mag_sq / (1.0 + mag_sq)) * (sb / (mag + 1e-8))


def kernel(u_hat, iters, bias):
    del iters
    B, J, I, N = u_hat.shape
    TI = min(128, I)
    TN = TI * N
    TI1 = min(512, I)
    TN1 = TI1 * N
    f32 = jnp.float32

    k0 = math.ceil(J * 0.5)
    k1 = math.ceil(k0 * 0.5)

    u2 = u_hat.reshape(B, J, I * N)

    m_idx = jnp.arange(TN, dtype=jnp.int32)
    S_mat = (m_idx[:, None] // N == jnp.arange(TI, dtype=jnp.int32)[None, :]).astype(f32)
    R_mat = (m_idx[:, None] % N == jnp.arange(N, dtype=jnp.int32)[None, :]).astype(f32)
    St_mat = S_mat.T
    Rt_mat = R_mat.T
    m1_idx = jnp.arange(TN1, dtype=jnp.int32)
    R1_mat = (m1_idx[:, None] % N == jnp.arange(N, dtype=jnp.int32)[None, :]).astype(f32)

    cparams = pltpu.CompilerParams(
        dimension_semantics=("parallel", "arbitrary"))

    s0 = pl.pallas_call(
        _pass1_body,
        grid=(B, I // TI1),
        in_specs=[
            pl.BlockSpec((1, J, TN1), lambda b, t: (b, 0, t)),
            pl.BlockSpec((TN1, N), lambda b, t: (0, 0)),
        ],
        out_specs=pl.BlockSpec((1, J, N), lambda b, t: (b, 0, 0)),
        out_shape=jax.ShapeDtypeStruct((B, J, N), f32),
        compiler_params=cparams,
    )(u2, R1_mat)

    v0 = _squash_bias(s0 * (1.0 / J), bias)

    s1, ent1, bvec1 = pl.pallas_call(
        lambda *refs: _pass2_body(k0, *refs),
        grid=(B, I // TI),
        in_specs=[
            pl.BlockSpec((1, J, TN), lambda b, t: (b, 0, t)),
            pl.BlockSpec((1, J, N), lambda b, t: (b, 0, 0)),
            pl.BlockSpec((N, TN), lambda b, t: (0, 0)),
            pl.BlockSpec((TN, TI), lambda b, t: (0, 0)),
            pl.BlockSpec((TI, TN), lambda b, t: (0, 0)),
            pl.BlockSpec((TN, N), lambda b, t: (0, 0)),
        ],
        out_specs=[
            pl.BlockSpec((1, J, N), lambda b, t: (b, 0, 0)),
            pl.BlockSpec((1, 8, TI), lambda b, t: (b, 0, 0)),
            pl.BlockSpec((1, J, TI), lambda b, t: (b, 0, t)),
        ],
        out_shape=[
            jax.ShapeDtypeStruct((B, J, N), f32),
            jax.ShapeDtypeStruct((B, 8, TI), f32),
            jax.ShapeDtypeStruct((B, J, I), f32),
        ],
        compiler_params=cparams,
    )(u2, v0, Rt_mat, S_mat, St_mat, R_mat)

    v1 = _squash_bias(s1, bias)

    s2, ent2 = pl.pallas_call(
        lambda *refs: _pass3_body(k1, *refs),
        grid=(B, I // TI),
        in_specs=[
            pl.BlockSpec((1, J, TN), lambda b, t: (b, 0, t)),
            pl.BlockSpec((1, J, N), lambda b, t: (b, 0, 0)),
            pl.BlockSpec((1, J, TI), lambda b, t: (b, 0, t)),
            pl.BlockSpec((N, TN), lambda b, t: (0, 0)),
            pl.BlockSpec((TN, TI), lambda b, t: (0, 0)),
            pl.BlockSpec((TI, TN), lambda b, t: (0, 0)),
            pl.BlockSpec((TN, N), lambda b, t: (0, 0)),
        ],
        out_specs=[
            pl.BlockSpec((1, J, N), lambda b, t: (b, 0, 0)),
            pl.BlockSpec((1, 8, TI), lambda b, t: (b, 0, 0)),
        ],
        out_shape=[
            jax.ShapeDtypeStruct((B, J, N), f32),
            jax.ShapeDtypeStruct((B, 8, TI), f32),
        ],
        compiler_params=cparams,
    )(u2, v1, bvec1, Rt_mat, S_mat, St_mat, R_mat)

    v2 = _squash_bias(s2, bias)

    e0 = jnp.full((B,), jnp.log(f32(J)), dtype=f32)
    e1 = jnp.sum(ent1[:, 0, :], axis=-1) * (1.0 / I)
    e2 = jnp.sum(ent2[:, 0, :], axis=-1) * (1.0 / I)
    entropy_layer = jnp.stack([e0, e1, e2], axis=1)
    return v2, entropy_layer
